# Initial kernel scaffold; baseline (speedup 1.0000x reference)
import jax
import jax.numpy as jnp
from jax.experimental import pallas as pl


def kernel(x, edge_index, W1, a_src1, a_dst1, b1, W2, a_src2, a_dst2, b2):
    raise NotImplementedError("write your pallas kernel here")


# speedup vs baseline: 8.5742x; 8.5742x over previous
import functools

import jax
import jax.numpy as jnp
from jax import lax
from jax.experimental import pallas as pl
from jax.experimental.pallas import tpu as pltpu
from jax.experimental.pallas import tpu_sc as plsc

NC, NS, L = 2, 16, 16
NW = NC * NS
CH = 128
HP = 128


def _proj_body(x_ref, w_ref, as_ref, ad_ref, h_ref, s_ref, d_ref):
    h = jnp.dot(x_ref[...], w_ref[...], preferred_element_type=jnp.float32)
    h_ref[...] = h
    s_ref[...] = jnp.sum(h * as_ref[...], axis=1, keepdims=True)
    d_ref[...] = jnp.sum(h * ad_ref[...], axis=1, keepdims=True)


def _project(x, W, a_src, a_dst, rb=1000):
    n, f = x.shape
    hd = W.shape[1]
    return pl.pallas_call(
        _proj_body,
        grid=(n // rb,),
        in_specs=[
            pl.BlockSpec((rb, f), lambda i: (i, 0)),
            pl.BlockSpec((f, hd), lambda i: (0, 0)),
            pl.BlockSpec((1, hd), lambda i: (0, 0)),
            pl.BlockSpec((1, hd), lambda i: (0, 0)),
        ],
        out_specs=[
            pl.BlockSpec((rb, hd), lambda i: (i, 0)),
            pl.BlockSpec((rb, 1), lambda i: (i, 0)),
            pl.BlockSpec((rb, 1), lambda i: (i, 0)),
        ],
        out_shape=[
            jax.ShapeDtypeStruct((n, hd), jnp.float32),
            jax.ShapeDtypeStruct((n, 1), jnp.float32),
            jax.ShapeDtypeStruct((n, 1), jnp.float32),
        ],
    )(x, W, a_src.reshape(1, -1), a_dst.reshape(1, -1))


def _mid_body(acc_ref, den_ref, b_ref, w_ref, as_ref, ad_ref,
              h_ref, s_ref, d_ref, *, hin, hd):
    num = acc_ref[0] + acc_ref[1]
    den = jnp.sum(den_ref[...], axis=0)
    h1 = jnp.maximum(num[:, :hin] / (den + 1e-16) + b_ref[...], 0.0)
    h = jnp.dot(h1, w_ref[...], preferred_element_type=jnp.float32)
    h_ref[:, :hd] = h
    if hd < HP:
        h_ref[:, hd:] = jnp.zeros_like(h_ref[:, hd:])
    s_ref[...] = jnp.sum(h * as_ref[...], axis=1, keepdims=True)
    d_ref[...] = jnp.sum(h * ad_ref[...], axis=1, keepdims=True)


def _mid(acc, den, b, W, a_src, a_dst, n, hin, rb=1000):
    hd = W.shape[1]
    return pl.pallas_call(
        functools.partial(_mid_body, hin=hin, hd=hd),
        grid=(n // rb,),
        in_specs=[
            pl.BlockSpec((2, rb, HP), lambda i: (0, i, 0)),
            pl.BlockSpec((NW, rb, 1), lambda i: (0, i, 0)),
            pl.BlockSpec((1, hin), lambda i: (0, 0)),
            pl.BlockSpec((hin, hd), lambda i: (0, 0)),
            pl.BlockSpec((1, hd), lambda i: (0, 0)),
            pl.BlockSpec((1, hd), lambda i: (0, 0)),
        ],
        out_specs=[
            pl.BlockSpec((rb, HP), lambda i: (i, 0)),
            pl.BlockSpec((rb, 1), lambda i: (i, 0)),
            pl.BlockSpec((rb, 1), lambda i: (i, 0)),
        ],
        out_shape=[
            jax.ShapeDtypeStruct((n, HP), jnp.float32),
            jax.ShapeDtypeStruct((n, 1), jnp.float32),
            jax.ShapeDtypeStruct((n, 1), jnp.float32),
        ],
    )(acc, den, b.reshape(1, -1), W, a_src.reshape(1, -1), a_dst.reshape(1, -1))


def _fin_body(acc_ref, den_ref, b_ref, z_ref, *, hin):
    num = acc_ref[0] + acc_ref[1]
    den = jnp.sum(den_ref[...], axis=0)
    z_ref[...] = jnp.tanh(num[:, :hin] / (den + 1e-16) + b_ref[...])


def _fin(acc, den, b, n, hin, rb=1000):
    return pl.pallas_call(
        functools.partial(_fin_body, hin=hin),
        grid=(n // rb,),
        in_specs=[
            pl.BlockSpec((2, rb, HP), lambda i: (0, i, 0)),
            pl.BlockSpec((NW, rb, 1), lambda i: (0, i, 0)),
            pl.BlockSpec((1, hin), lambda i: (0, 0)),
        ],
        out_specs=pl.BlockSpec((rb, hin), lambda i: (i, 0)),
        out_shape=jax.ShapeDtypeStruct((n, hin), jnp.float32),
    )(acc, den, b.reshape(1, -1))


def _dec_body(zr_ref, zc_ref, o_ref):
    g = lax.dot_general(zr_ref[...], zc_ref[...], (((1,), (1,)), ((), ())),
                        preferred_element_type=jnp.float32)
    o_ref[...] = jax.nn.sigmoid(g)


def _decode(z, rb=1024, cb=1024):
    n, hd = z.shape
    return pl.pallas_call(
        _dec_body,
        grid=(pl.cdiv(n, rb), pl.cdiv(n, cb)),
        in_specs=[
            pl.BlockSpec((rb, hd), lambda i, j: (i, 0)),
            pl.BlockSpec((cb, hd), lambda i, j: (j, 0)),
        ],
        out_specs=pl.BlockSpec((rb, cb), lambda i, j: (i, j)),
        out_shape=jax.ShapeDtypeStruct((n, n), jnp.float32),
    )(z, z)


def _sc_edge(h, src2d, dst2d, s_pad, d_pad, npad):
    cpt = src2d.shape[0] // NW
    rpt = npad // NS
    zc = rpt // CH
    mesh = plsc.VectorSubcoreMesh(
        core_axis_name="c", subcore_axis_name="s", num_cores=NC, num_subcores=NS)

    @functools.partial(
        pl.kernel,
        out_type=[
            jax.ShapeDtypeStruct((NC, npad, HP), jnp.float32),
            jax.ShapeDtypeStruct((NW, npad), jnp.float32),
        ],
        mesh=mesh,
        compiler_params=pltpu.CompilerParams(needs_layout_passes=False),
        scratch_types=[
            pltpu.VMEM((npad,), jnp.float32),
            pltpu.VMEM((npad,), jnp.float32),
            pltpu.VMEM((npad,), jnp.float32),
            pltpu.VMEM((CH,), jnp.int32),
            pltpu.VMEM((CH,), jnp.int32),
            pltpu.VMEM((CH,), jnp.float32),
            pltpu.VMEM((CH, HP), jnp.float32),
            pltpu.VMEM_SHARED((npad, HP), jnp.float32),
            pltpu.SemaphoreType.DMA,
        ],
    )
    def k(h_hbm, src_hbm, dst_hbm, sv_hbm, dv_hbm, num_hbm, den_hbm,
          s_v, d_v, dloc, srcb, dstb, wb, gb, acc, sem):
        c = lax.axis_index("c")
        sid = lax.axis_index("s")
        wid = c * NS + sid
        pltpu.sync_copy(sv_hbm, s_v)
        pltpu.sync_copy(dv_hbm, d_v)

        zero16 = jnp.zeros((L,), jnp.float32)

        def zrow(j, carry):
            for kk in range(HP // L):
                gb[j, pl.ds(kk * L, L)] = zero16
            return carry

        lax.fori_loop(0, CH, zrow, 0)

        def zden(j, carry):
            dloc[pl.ds(j * L, L)] = zero16
            return carry

        lax.fori_loop(0, npad // L, zden, 0)

        base = sid * rpt
        for i in range(zc):
            pltpu.sync_copy(gb, acc.at[pl.ds(base + i * CH, CH)])
        plsc.subcore_barrier()

        def chunk(i, carry):
            r = wid * cpt + i
            pltpu.sync_copy(src_hbm.at[r], srcb)
            pltpu.sync_copy(dst_hbm.at[r], dstb)
            pltpu.async_copy(h_hbm.at[srcb], gb, sem).wait()
            for g in range(CH // L):
                dv = dstb[pl.ds(g * L, L)]
                e = (plsc.load_gather(s_v, [srcb[pl.ds(g * L, L)]])
                     + plsc.load_gather(d_v, [dv]))
                e = jnp.where(e >= 0.0, e, 0.2 * e)
                w = jnp.exp(jnp.minimum(e, 75.0))
                wb[pl.ds(g * L, L)] = w
                plsc.addupdate_scatter(dloc, [dv], w)

            def edge(j, ecarry):
                wspl = plsc.load_gather(wb, [jnp.zeros((L,), jnp.int32) + j])
                for kk in range(HP // L):
                    gb[j, pl.ds(kk * L, L)] = gb[j, pl.ds(kk * L, L)] * wspl
                return ecarry

            lax.fori_loop(0, CH, edge, 0)
            pltpu.async_copy(gb, acc.at[dstb], sem, add=True).wait()
            return carry

        lax.fori_loop(0, cpt, chunk, 0)

        pltpu.sync_copy(dloc, den_hbm.at[wid])
        plsc.subcore_barrier()

        for i in range(zc):
            pltpu.sync_copy(acc.at[pl.ds(base + i * CH, CH)], gb)
            pltpu.sync_copy(gb, num_hbm.at[c, pl.ds(base + i * CH, CH)])

    return k(h, src2d, dst2d, s_pad, d_pad)


def kernel(x, edge_index, W1, a_src1, a_dst1, b1, W2, a_src2, a_dst2, b2):
    n = x.shape[0]
    e = edge_index.shape[1]
    h1d = W1.shape[1]
    h2d = W2.shape[1]
    npad = ((n + 1 + NS * CH - 1) // (NS * CH)) * (NS * CH)
    ep = ((e + NW * CH - 1) // (NW * CH)) * (NW * CH)
    src = jnp.concatenate(
        [edge_index[0], jnp.zeros((ep - e,), jnp.int32)]).reshape(ep // CH, CH)
    dst = jnp.concatenate(
        [edge_index[1], jnp.full((ep - e,), n, jnp.int32)]).reshape(ep // CH, CH)

    h1, s1, d1 = _project(x, W1, a_src1, a_dst1)
    s1p = jnp.pad(s1[:, 0], (0, npad - n))
    d1p = jnp.pad(d1[:, 0], (0, npad - n))
    num1, den1 = _sc_edge(h1, src, dst, s1p, d1p, npad)

    h2, s2, d2 = _mid(num1, den1.reshape(NW, npad, 1), b1, W2, a_src2, a_dst2,
                      n, h1d)
    s2p = jnp.pad(s2[:, 0], (0, npad - n))
    d2p = jnp.pad(d2[:, 0], (0, npad - n))
    num2, den2 = _sc_edge(h2, src, dst, s2p, d2p, npad)

    z = _fin(num2, den2.reshape(NW, npad, 1), b2, n, h2d)
    adj = _decode(z)
    return (adj, z)


# speedup vs baseline: 11.8381x; 1.3807x over previous
import functools

import jax
import jax.numpy as jnp
from jax import lax
from jax.experimental import pallas as pl
from jax.experimental.pallas import tpu as pltpu
from jax.experimental.pallas import tpu_sc as plsc

NC, NS, L = 2, 16, 16
NW = NC * NS
CH = 128
HP = 128


def _proj_body(x_ref, w_ref, as_ref, ad_ref, h_ref, s_ref, d_ref):
    h = jnp.dot(x_ref[...], w_ref[...], preferred_element_type=jnp.float32)
    h_ref[...] = h
    s_ref[...] = jnp.sum(h * as_ref[...], axis=1, keepdims=True)
    d_ref[...] = jnp.sum(h * ad_ref[...], axis=1, keepdims=True)


def _project(x, W, a_src, a_dst, rb=1000):
    n, f = x.shape
    hd = W.shape[1]
    return pl.pallas_call(
        _proj_body,
        grid=(n // rb,),
        in_specs=[
            pl.BlockSpec((rb, f), lambda i: (i, 0)),
            pl.BlockSpec((f, hd), lambda i: (0, 0)),
            pl.BlockSpec((1, hd), lambda i: (0, 0)),
            pl.BlockSpec((1, hd), lambda i: (0, 0)),
        ],
        out_specs=[
            pl.BlockSpec((rb, hd), lambda i: (i, 0)),
            pl.BlockSpec((rb, 1), lambda i: (i, 0)),
            pl.BlockSpec((rb, 1), lambda i: (i, 0)),
        ],
        out_shape=[
            jax.ShapeDtypeStruct((n, hd), jnp.float32),
            jax.ShapeDtypeStruct((n, 1), jnp.float32),
            jax.ShapeDtypeStruct((n, 1), jnp.float32),
        ],
    )(x, W, a_src.reshape(1, -1), a_dst.reshape(1, -1))


def _mid_body(acc_ref, den_ref, b_ref, w_ref, as_ref, ad_ref,
              h_ref, s_ref, d_ref, *, hin, hd):
    num = acc_ref[0] + acc_ref[1]
    den = jnp.sum(den_ref[...], axis=0)
    h1 = jnp.maximum(num[:, :hin] / (den + 1e-16) + b_ref[...], 0.0)
    h = jnp.dot(h1, w_ref[...], preferred_element_type=jnp.float32)
    h_ref[:, :hd] = h
    if hd < HP:
        h_ref[:, hd:] = jnp.zeros_like(h_ref[:, hd:])
    s_ref[...] = jnp.sum(h * as_ref[...], axis=1, keepdims=True)
    d_ref[...] = jnp.sum(h * ad_ref[...], axis=1, keepdims=True)


def _mid(acc, den, b, W, a_src, a_dst, n, hin, rb=1000):
    hd = W.shape[1]
    return pl.pallas_call(
        functools.partial(_mid_body, hin=hin, hd=hd),
        grid=(n // rb,),
        in_specs=[
            pl.BlockSpec((2, rb, HP), lambda i: (0, i, 0)),
            pl.BlockSpec((NW, rb, 1), lambda i: (0, i, 0)),
            pl.BlockSpec((1, hin), lambda i: (0, 0)),
            pl.BlockSpec((hin, hd), lambda i: (0, 0)),
            pl.BlockSpec((1, hd), lambda i: (0, 0)),
            pl.BlockSpec((1, hd), lambda i: (0, 0)),
        ],
        out_specs=[
            pl.BlockSpec((rb, HP), lambda i: (i, 0)),
            pl.BlockSpec((rb, 1), lambda i: (i, 0)),
            pl.BlockSpec((rb, 1), lambda i: (i, 0)),
        ],
        out_shape=[
            jax.ShapeDtypeStruct((n, HP), jnp.float32),
            jax.ShapeDtypeStruct((n, 1), jnp.float32),
            jax.ShapeDtypeStruct((n, 1), jnp.float32),
        ],
    )(acc, den, b.reshape(1, -1), W, a_src.reshape(1, -1), a_dst.reshape(1, -1))


def _fin_body(acc_ref, den_ref, b_ref, z_ref, *, hin):
    num = acc_ref[0] + acc_ref[1]
    den = jnp.sum(den_ref[...], axis=0)
    z_ref[...] = jnp.tanh(num[:, :hin] / (den + 1e-16) + b_ref[...])


def _fin(acc, den, b, n, hin, rb=1000):
    return pl.pallas_call(
        functools.partial(_fin_body, hin=hin),
        grid=(n // rb,),
        in_specs=[
            pl.BlockSpec((2, rb, HP), lambda i: (0, i, 0)),
            pl.BlockSpec((NW, rb, 1), lambda i: (0, i, 0)),
            pl.BlockSpec((1, hin), lambda i: (0, 0)),
        ],
        out_specs=pl.BlockSpec((rb, hin), lambda i: (i, 0)),
        out_shape=jax.ShapeDtypeStruct((n, hin), jnp.float32),
    )(acc, den, b.reshape(1, -1))


def _dec_body(zr_ref, zc_ref, o_ref):
    g = lax.dot_general(zr_ref[...], zc_ref[...], (((1,), (1,)), ((), ())),
                        preferred_element_type=jnp.float32)
    o_ref[...] = jax.nn.sigmoid(g)


def _decode(z, rb=1024, cb=1024):
    n, hd = z.shape
    return pl.pallas_call(
        _dec_body,
        grid=(pl.cdiv(n, rb), pl.cdiv(n, cb)),
        in_specs=[
            pl.BlockSpec((rb, hd), lambda i, j: (i, 0)),
            pl.BlockSpec((cb, hd), lambda i, j: (j, 0)),
        ],
        out_specs=pl.BlockSpec((rb, cb), lambda i, j: (i, j)),
        out_shape=jax.ShapeDtypeStruct((n, n), jnp.float32),
    )(z, z)


EC = 64


def _sc_weights(src2d, dst2d, s_pad, d_pad, npad):
    cpt = src2d.shape[0] // NW
    mesh = plsc.VectorSubcoreMesh(
        core_axis_name="c", subcore_axis_name="s", num_cores=NC, num_subcores=NS)

    @functools.partial(
        pl.kernel,
        out_type=[
            jax.ShapeDtypeStruct(src2d.shape, jnp.float32),
            jax.ShapeDtypeStruct((NW, npad), jnp.float32),
        ],
        mesh=mesh,
        compiler_params=pltpu.CompilerParams(needs_layout_passes=False),
        scratch_types=[
            pltpu.VMEM((npad,), jnp.float32),
            pltpu.VMEM((npad,), jnp.float32),
            pltpu.VMEM((npad,), jnp.float32),
            pltpu.VMEM((cpt, EC), jnp.int32),
            pltpu.VMEM((cpt, EC), jnp.int32),
            pltpu.VMEM((cpt, EC), jnp.float32),
        ],
    )
    def k(src_hbm, dst_hbm, sv_hbm, dv_hbm, w_hbm, den_hbm,
          s_v, d_v, dloc, srca, dsta, wa):
        c = lax.axis_index("c")
        sid = lax.axis_index("s")
        wid = c * NS + sid
        pltpu.sync_copy(src_hbm.at[pl.ds(wid * cpt, cpt)], srca)
        pltpu.sync_copy(dst_hbm.at[pl.ds(wid * cpt, cpt)], dsta)
        pltpu.sync_copy(sv_hbm, s_v)
        pltpu.sync_copy(dv_hbm, d_v)
        zero16 = jnp.zeros((L,), jnp.float32)

        def zden(j, carry):
            dloc[pl.ds(j * L, L)] = zero16
            return carry

        lax.fori_loop(0, npad // L, zden, 0)

        def wchunk(i, carry):
            for g in range(EC // L):
                dv = dsta[i, pl.ds(g * L, L)]
                e = (plsc.load_gather(s_v, [srca[i, pl.ds(g * L, L)]])
                     + plsc.load_gather(d_v, [dv]))
                e = jnp.where(e >= 0.0, e, 0.2 * e)
                w = jnp.exp(jnp.minimum(e, 75.0))
                wa[i, pl.ds(g * L, L)] = w
                plsc.addupdate_scatter(dloc, [dv], w)
            return carry

        lax.fori_loop(0, cpt, wchunk, 0)
        pltpu.sync_copy(wa, w_hbm.at[pl.ds(wid * cpt, cpt)])
        pltpu.sync_copy(dloc, den_hbm.at[wid])

    return k(src2d, dst2d, s_pad, d_pad)


def _sc_aggregate(h, src1d, dst2d, w1d, rpad):
    cpt = dst2d.shape[0] // NW
    rpt = rpad // NS
    mesh = plsc.VectorSubcoreMesh(
        core_axis_name="c", subcore_axis_name="s", num_cores=NC, num_subcores=NS)

    @functools.partial(
        pl.kernel,
        out_type=jax.ShapeDtypeStruct((NC, rpad, HP), jnp.float32),
        mesh=mesh,
        compiler_params=pltpu.CompilerParams(needs_layout_passes=False),
        scratch_types=[
            pltpu.VMEM((cpt * EC,), jnp.int32),
            pltpu.VMEM((cpt, EC), jnp.int32),
            pltpu.VMEM((2 * EC,), jnp.float32),
            pltpu.VMEM((2, EC, HP), jnp.float32),
            pltpu.VMEM((2, EC, HP), jnp.float32),
            pltpu.VMEM_SHARED((rpad, HP), jnp.float32),
            pltpu.SemaphoreType.DMA,
            pltpu.SemaphoreType.DMA,
            pltpu.SemaphoreType.DMA,
            pltpu.SemaphoreType.DMA,
            pltpu.SemaphoreType.DMA,
            pltpu.SemaphoreType.DMA,
        ],
    )
    def k(h_hbm, src_hbm, dst_hbm, w_hbm, num_hbm,
          srca, dsta, wb, gb, sb, acc, sg0, sg1, ss0, ss1, sw0, sw1):
        c = lax.axis_index("c")
        sid = lax.axis_index("s")
        wid = c * NS + sid
        sgs = (sg0, sg1)
        sss = (ss0, ss1)
        sws = (sw0, sw1)
        zero16 = jnp.zeros((L,), jnp.float32)
        pltpu.sync_copy(src_hbm.at[pl.ds(wid * cpt * EC, cpt * EC)], srca)
        pltpu.sync_copy(dst_hbm.at[pl.ds(wid * cpt, cpt)], dsta)

        base = sid * rpt

        def zrow(j, carry):
            for kk in range(HP // L):
                sb[0, j, pl.ds(kk * L, L)] = zero16
            return carry

        lax.fori_loop(0, EC, zrow, 0)
        off = 0
        while off < rpt:
            step = min(EC, rpt - off)
            pltpu.sync_copy(sb.at[0, pl.ds(0, step)],
                            acc.at[pl.ds(base + off, step)])
            off += step
        plsc.subcore_barrier()

        def g_start(ci, b):
            pltpu.async_copy(h_hbm.at[srca.at[pl.ds(ci * EC, EC)]],
                             gb.at[b], sgs[b])

        def g_wait(b):
            pltpu.make_async_copy(h_hbm.at[srca.at[pl.ds(0, EC)]],
                                  gb.at[b], sgs[b]).wait()

        def w_start(ci, b):
            pltpu.async_copy(w_hbm.at[pl.ds((wid * cpt + ci) * EC, EC)],
                             wb.at[pl.ds(b * EC, EC)], sws[b])

        def w_wait(b):
            pltpu.make_async_copy(w_hbm.at[pl.ds(0, EC)],
                                  wb.at[pl.ds(b * EC, EC)], sws[b]).wait()

        def s_start(ci, b):
            pltpu.async_copy(sb.at[b], acc.at[dsta.at[ci]], sss[b], add=True)

        def s_wait(b):
            pltpu.make_async_copy(sb.at[b], acc.at[dsta.at[0]], sss[b]).wait()

        def scale(b):
            def edge(j, ecarry):
                wspl = plsc.load_gather(
                    wb, [jnp.zeros((L,), jnp.int32) + (j + b * EC)])
                for kk in range(HP // L):
                    sb[b, j, pl.ds(kk * L, L)] = gb[b, j, pl.ds(kk * L, L)] * wspl
                return ecarry

            lax.fori_loop(0, EC, edge, 0)

        for b in range(2):
            g_start(b, b)
            w_start(b, b)
        for b in range(2):
            g_wait(b)
            w_wait(b)
            scale(b)
            s_start(b, b)
            g_start(b + 2, b)
            w_start(b + 2, b)

        def steady(o, carry):
            for b in range(2):
                ci = 2 + 2 * o + b
                g_wait(b)
                w_wait(b)
                s_wait(b)
                scale(b)
                s_start(ci, b)
                g_start(ci + 2, b)
                w_start(ci + 2, b)
            return carry

        lax.fori_loop(0, (cpt - 4) // 2, steady, 0)
        for b in range(2):
            ci = cpt - 2 + b
            g_wait(b)
            w_wait(b)
            s_wait(b)
            scale(b)
            s_start(ci, b)
        for b in range(2):
            s_wait(b)
        plsc.subcore_barrier()

        pltpu.sync_copy(acc.at[pl.ds(base, rpt)],
                        num_hbm.at[c, pl.ds(base, rpt)])

    return k(h, src1d, dst2d, w1d)


def kernel(x, edge_index, W1, a_src1, a_dst1, b1, W2, a_src2, a_dst2, b2):
    n = x.shape[0]
    e = edge_index.shape[1]
    h1d = W1.shape[1]
    h2d = W2.shape[1]
    npad = ((n + 1 + NS * CH - 1) // (NS * CH)) * (NS * CH)
    rpad = ((n + 1 + NS * 8 - 1) // (NS * 8)) * (NS * 8)
    ep = ((e + NW * EC * 2 - 1) // (NW * EC * 2)) * (NW * EC * 2)
    src = jnp.concatenate(
        [edge_index[0], jnp.zeros((ep - e,), jnp.int32)]).reshape(ep // EC, EC)
    dst = jnp.concatenate(
        [edge_index[1], jnp.full((ep - e,), n, jnp.int32)]).reshape(ep // EC, EC)

    h1, s1, d1 = _project(x, W1, a_src1, a_dst1)
    s1p = jnp.pad(s1[:, 0], (0, npad - n))
    d1p = jnp.pad(d1[:, 0], (0, npad - n))
    w1, den1 = _sc_weights(src, dst, s1p, d1p, npad)
    num1 = _sc_aggregate(h1, src.reshape(-1), dst, w1.reshape(-1), rpad)

    h2, s2, d2 = _mid(num1, den1.reshape(NW, npad, 1), b1, W2, a_src2, a_dst2,
                      n, h1d)
    s2p = jnp.pad(s2[:, 0], (0, npad - n))
    d2p = jnp.pad(d2[:, 0], (0, npad - n))
    w2, den2 = _sc_weights(src, dst, s2p, d2p, npad)
    num2 = _sc_aggregate(h2, src.reshape(-1), dst, w2.reshape(-1), rpad)

    z = _fin(num2, den2.reshape(NW, npad, 1), b2, n, h2d)
    adj = _decode(z)
    return (adj, z)


# speedup vs baseline: 13.4017x; 1.1321x over previous
import functools

import jax
import jax.numpy as jnp
from jax import lax
from jax.experimental import pallas as pl
from jax.experimental.pallas import tpu as pltpu
from jax.experimental.pallas import tpu_sc as plsc

NC, NS, L = 2, 16, 16
NW = NC * NS
CH = 128
HP = 128


def _proj_body(x_ref, w_ref, as_ref, ad_ref, h_ref, s_ref, d_ref):
    h = jnp.dot(x_ref[...], w_ref[...], preferred_element_type=jnp.float32)
    h_ref[...] = h
    s_ref[...] = jnp.sum(h * as_ref[...], axis=1, keepdims=True)
    d_ref[...] = jnp.sum(h * ad_ref[...], axis=1, keepdims=True)


def _project(x, W, a_src, a_dst, rb=1000):
    n, f = x.shape
    hd = W.shape[1]
    return pl.pallas_call(
        _proj_body,
        grid=(n // rb,),
        in_specs=[
            pl.BlockSpec((rb, f), lambda i: (i, 0)),
            pl.BlockSpec((f, hd), lambda i: (0, 0)),
            pl.BlockSpec((1, hd), lambda i: (0, 0)),
            pl.BlockSpec((1, hd), lambda i: (0, 0)),
        ],
        out_specs=[
            pl.BlockSpec((rb, hd), lambda i: (i, 0)),
            pl.BlockSpec((rb, 1), lambda i: (i, 0)),
            pl.BlockSpec((rb, 1), lambda i: (i, 0)),
        ],
        out_shape=[
            jax.ShapeDtypeStruct((n, hd), jnp.float32),
            jax.ShapeDtypeStruct((n, 1), jnp.float32),
            jax.ShapeDtypeStruct((n, 1), jnp.float32),
        ],
    )(x, W, a_src.reshape(1, -1), a_dst.reshape(1, -1))


def _mid_body(acc_ref, den_ref, b_ref, w_ref, as_ref, ad_ref,
              h_ref, s_ref, d_ref, *, hin, hd):
    num = acc_ref[0] + acc_ref[1]
    den = jnp.sum(den_ref[...], axis=0)
    h1 = jnp.maximum(num[:, :hin] / (den + 1e-16) + b_ref[...], 0.0)
    h = jnp.dot(h1, w_ref[...], preferred_element_type=jnp.float32)
    h_ref[...] = h
    s_ref[...] = jnp.sum(h * as_ref[...], axis=1, keepdims=True)
    d_ref[...] = jnp.sum(h * ad_ref[...], axis=1, keepdims=True)


def _mid(acc, den, b, W, a_src, a_dst, n, hin, rb=1000):
    hd = W.shape[1]
    return pl.pallas_call(
        functools.partial(_mid_body, hin=hin, hd=hd),
        grid=(n // rb,),
        in_specs=[
            pl.BlockSpec((2, rb, HP), lambda i: (0, i, 0)),
            pl.BlockSpec((NW, rb, 1), lambda i: (0, i, 0)),
            pl.BlockSpec((1, hin), lambda i: (0, 0)),
            pl.BlockSpec((hin, hd), lambda i: (0, 0)),
            pl.BlockSpec((1, hd), lambda i: (0, 0)),
            pl.BlockSpec((1, hd), lambda i: (0, 0)),
        ],
        out_specs=[
            pl.BlockSpec((rb, hd), lambda i: (i, 0)),
            pl.BlockSpec((rb, 1), lambda i: (i, 0)),
            pl.BlockSpec((rb, 1), lambda i: (i, 0)),
        ],
        out_shape=[
            jax.ShapeDtypeStruct((n, hd), jnp.float32),
            jax.ShapeDtypeStruct((n, 1), jnp.float32),
            jax.ShapeDtypeStruct((n, 1), jnp.float32),
        ],
    )(acc, den, b.reshape(1, -1), W, a_src.reshape(1, -1), a_dst.reshape(1, -1))


def _fin_body(acc_ref, den_ref, b_ref, z_ref, *, hin):
    num = acc_ref[0] + acc_ref[1]
    den = jnp.sum(den_ref[...], axis=0)
    z_ref[...] = jnp.tanh(num[:, :hin] / (den + 1e-16) + b_ref[...])


def _fin(acc, den, b, n, hin, rb=1000):
    hp = acc.shape[2]
    return pl.pallas_call(
        functools.partial(_fin_body, hin=hin),
        grid=(n // rb,),
        in_specs=[
            pl.BlockSpec((2, rb, hp), lambda i: (0, i, 0)),
            pl.BlockSpec((NW, rb, 1), lambda i: (0, i, 0)),
            pl.BlockSpec((1, hin), lambda i: (0, 0)),
        ],
        out_specs=pl.BlockSpec((rb, hin), lambda i: (i, 0)),
        out_shape=jax.ShapeDtypeStruct((n, hin), jnp.float32),
    )(acc, den, b.reshape(1, -1))


def _dec_body(zr_ref, zc_ref, o_ref):
    g = lax.dot_general(zr_ref[...], zc_ref[...], (((1,), (1,)), ((), ())),
                        preferred_element_type=jnp.float32)
    o_ref[...] = jax.nn.sigmoid(g)


def _decode(z, rb=1024, cb=1024):
    n, hd = z.shape
    return pl.pallas_call(
        _dec_body,
        grid=(pl.cdiv(n, rb), pl.cdiv(n, cb)),
        in_specs=[
            pl.BlockSpec((rb, hd), lambda i, j: (i, 0)),
            pl.BlockSpec((cb, hd), lambda i, j: (j, 0)),
        ],
        out_specs=pl.BlockSpec((rb, cb), lambda i, j: (i, j)),
        out_shape=jax.ShapeDtypeStruct((n, n), jnp.float32),
    )(z, z)


EC = 64


def _sc_weights(src2d, dst2d, s_pad, d_pad, npad):
    cpt = src2d.shape[0] // NW
    mesh = plsc.VectorSubcoreMesh(
        core_axis_name="c", subcore_axis_name="s", num_cores=NC, num_subcores=NS)

    @functools.partial(
        pl.kernel,
        out_type=[
            jax.ShapeDtypeStruct(src2d.shape, jnp.float32),
            jax.ShapeDtypeStruct((NW, npad), jnp.float32),
        ],
        mesh=mesh,
        compiler_params=pltpu.CompilerParams(needs_layout_passes=False),
        scratch_types=[
            pltpu.VMEM((npad,), jnp.float32),
            pltpu.VMEM((npad,), jnp.float32),
            pltpu.VMEM((npad,), jnp.float32),
            pltpu.VMEM((cpt, EC), jnp.int32),
            pltpu.VMEM((cpt, EC), jnp.int32),
            pltpu.VMEM((cpt, EC), jnp.float32),
        ],
    )
    def k(src_hbm, dst_hbm, sv_hbm, dv_hbm, w_hbm, den_hbm,
          s_v, d_v, dloc, srca, dsta, wa):
        c = lax.axis_index("c")
        sid = lax.axis_index("s")
        wid = c * NS + sid
        pltpu.sync_copy(src_hbm.at[pl.ds(wid * cpt, cpt)], srca)
        pltpu.sync_copy(dst_hbm.at[pl.ds(wid * cpt, cpt)], dsta)
        pltpu.sync_copy(sv_hbm, s_v)
        pltpu.sync_copy(dv_hbm, d_v)
        zero16 = jnp.zeros((L,), jnp.float32)

        def zden(j, carry):
            dloc[pl.ds(j * L, L)] = zero16
            return carry

        lax.fori_loop(0, npad // L, zden, 0)

        def wchunk(i, carry):
            for g in range(EC // L):
                dv = dsta[i, pl.ds(g * L, L)]
                e = (plsc.load_gather(s_v, [srca[i, pl.ds(g * L, L)]])
                     + plsc.load_gather(d_v, [dv]))
                e = jnp.where(e >= 0.0, e, 0.2 * e)
                w = jnp.exp(jnp.minimum(e, 75.0))
                wa[i, pl.ds(g * L, L)] = w
                plsc.addupdate_scatter(dloc, [dv], w)
            return carry

        lax.fori_loop(0, cpt, wchunk, 0)
        pltpu.sync_copy(wa, w_hbm.at[pl.ds(wid * cpt, cpt)])
        pltpu.sync_copy(dloc, den_hbm.at[wid])

    return k(src2d, dst2d, s_pad, d_pad)


def _sc_aggregate(h, src1d, dst2d, w1d, rpad, hp, ec):
    cpt = dst2d.shape[0] // NW
    rpt = rpad // NS
    mesh = plsc.VectorSubcoreMesh(
        core_axis_name="c", subcore_axis_name="s", num_cores=NC, num_subcores=NS)

    @functools.partial(
        pl.kernel,
        out_type=jax.ShapeDtypeStruct((NC, rpad, hp), jnp.float32),
        mesh=mesh,
        compiler_params=pltpu.CompilerParams(needs_layout_passes=False,
                                             use_tc_tiling_on_sc=False),
        scratch_types=[
            pltpu.VMEM((cpt * ec,), jnp.int32),
            pltpu.VMEM((cpt, ec), jnp.int32),
            pltpu.VMEM((2 * ec,), jnp.float32),
            pltpu.VMEM((2, ec, hp), jnp.float32),
            pltpu.VMEM((2, ec, hp), jnp.float32),
            pltpu.VMEM_SHARED((rpad, hp), jnp.float32),
            pltpu.SemaphoreType.DMA,
            pltpu.SemaphoreType.DMA,
            pltpu.SemaphoreType.DMA,
            pltpu.SemaphoreType.DMA,
            pltpu.SemaphoreType.DMA,
            pltpu.SemaphoreType.DMA,
        ],
    )
    def k(h_hbm, src_hbm, dst_hbm, w_hbm, num_hbm,
          srca, dsta, wb, gb, sb, acc, sg0, sg1, ss0, ss1, sw0, sw1):
        c = lax.axis_index("c")
        sid = lax.axis_index("s")
        wid = c * NS + sid
        sgs = (sg0, sg1)
        sss = (ss0, ss1)
        sws = (sw0, sw1)
        zero16 = jnp.zeros((L,), jnp.float32)
        pltpu.sync_copy(src_hbm.at[pl.ds(wid * cpt * ec, cpt * ec)], srca)
        pltpu.sync_copy(dst_hbm.at[pl.ds(wid * cpt, cpt)], dsta)

        base = sid * rpt

        def zrow(j, carry):
            for kk in range(hp // L):
                sb[0, j, pl.ds(kk * L, L)] = zero16
            return carry

        lax.fori_loop(0, ec, zrow, 0)
        off = 0
        while off < rpt:
            step = min(ec, rpt - off)
            pltpu.sync_copy(sb.at[0, pl.ds(0, step)],
                            acc.at[pl.ds(base + off, step)])
            off += step
        plsc.subcore_barrier()

        def g_start(ci, b):
            pltpu.async_copy(h_hbm.at[srca.at[pl.ds(ci * ec, ec)]],
                             gb.at[b], sgs[b])

        def g_wait(b):
            pltpu.make_async_copy(h_hbm.at[srca.at[pl.ds(0, ec)]],
                                  gb.at[b], sgs[b]).wait()

        def w_start(ci, b):
            pltpu.async_copy(w_hbm.at[pl.ds((wid * cpt + ci) * ec, ec)],
                             wb.at[pl.ds(b * ec, ec)], sws[b])

        def w_wait(b):
            pltpu.make_async_copy(w_hbm.at[pl.ds(0, ec)],
                                  wb.at[pl.ds(b * ec, ec)], sws[b]).wait()

        def s_start(ci, b):
            pltpu.async_copy(sb.at[b], acc.at[dsta.at[ci]], sss[b], add=True)

        def s_wait(b):
            pltpu.make_async_copy(sb.at[b], acc.at[dsta.at[0]], sss[b]).wait()

        def scale(b):
            def edge(j, ecarry):
                wspl = plsc.load_gather(
                    wb, [jnp.zeros((L,), jnp.int32) + (j + b * ec)])
                for kk in range(hp // L):
                    sb[b, j, pl.ds(kk * L, L)] = gb[b, j, pl.ds(kk * L, L)] * wspl
                return ecarry

            lax.fori_loop(0, ec, edge, 0)

        for b in range(2):
            g_start(b, b)
            w_start(b, b)
        for b in range(2):
            g_wait(b)
            w_wait(b)
            scale(b)
            s_start(b, b)
            g_start(b + 2, b)
            w_start(b + 2, b)

        def steady(o, carry):
            for b in range(2):
                ci = 2 + 2 * o + b
                g_wait(b)
                w_wait(b)
                s_wait(b)
                scale(b)
                s_start(ci, b)
                g_start(ci + 2, b)
                w_start(ci + 2, b)
            return carry

        lax.fori_loop(0, (cpt - 4) // 2, steady, 0)
        for b in range(2):
            ci = cpt - 2 + b
            g_wait(b)
            w_wait(b)
            s_wait(b)
            scale(b)
            s_start(ci, b)
        for b in range(2):
            s_wait(b)
        plsc.subcore_barrier()

        pltpu.sync_copy(acc.at[pl.ds(base, rpt)],
                        num_hbm.at[c, pl.ds(base, rpt)])

    return k(h, src1d, dst2d, w1d)


def kernel(x, edge_index, W1, a_src1, a_dst1, b1, W2, a_src2, a_dst2, b2):
    n = x.shape[0]
    e = edge_index.shape[1]
    h1d = W1.shape[1]
    h2d = W2.shape[1]
    npad = ((n + 1 + NS * CH - 1) // (NS * CH)) * (NS * CH)
    rpad = ((n + 1 + NS * 8 - 1) // (NS * 8)) * (NS * 8)
    ep = ((e + NW * EC * 2 - 1) // (NW * EC * 2)) * (NW * EC * 2)
    src = jnp.concatenate(
        [edge_index[0], jnp.zeros((ep - e,), jnp.int32)]).reshape(ep // EC, EC)
    dst = jnp.concatenate(
        [edge_index[1], jnp.full((ep - e,), n, jnp.int32)]).reshape(ep // EC, EC)

    h1, s1, d1 = _project(x, W1, a_src1, a_dst1)
    s1p = jnp.pad(s1[:, 0], (0, npad - n))
    d1p = jnp.pad(d1[:, 0], (0, npad - n))
    w1, den1 = _sc_weights(src, dst, s1p, d1p, npad)
    num1 = _sc_aggregate(h1, src.reshape(-1), dst, w1.reshape(-1), rpad,
                         h1d, EC)

    h2, s2, d2 = _mid(num1, den1.reshape(NW, npad, 1), b1, W2, a_src2, a_dst2,
                      n, h1d)
    s2p = jnp.pad(s2[:, 0], (0, npad - n))
    d2p = jnp.pad(d2[:, 0], (0, npad - n))
    w2, den2 = _sc_weights(src, dst, s2p, d2p, npad)
    num2 = _sc_aggregate(h2, src.reshape(-1), dst.reshape(ep // (2 * EC), 2 * EC),
                         w2.reshape(-1), rpad, h2d, 2 * EC)

    z = _fin(num2, den2.reshape(NW, npad, 1), b2, n, h2d)
    adj = _decode(z)
    return (adj, z)


# speedup vs baseline: 15.5714x; 1.1619x over previous
import functools

import jax
import jax.numpy as jnp
from jax import lax
from jax.experimental import pallas as pl
from jax.experimental.pallas import tpu as pltpu
from jax.experimental.pallas import tpu_sc as plsc

NC, NS, L = 2, 16, 16
NW = NC * NS
CH = 128
HP = 128


def _proj_body(x_ref, w_ref, as_ref, ad_ref, h_ref, s_ref, d_ref):
    h = jnp.dot(x_ref[...], w_ref[...], preferred_element_type=jnp.float32)
    h_ref[...] = h
    s_ref[...] = jnp.sum(h * as_ref[...], axis=1, keepdims=True)
    d_ref[...] = jnp.sum(h * ad_ref[...], axis=1, keepdims=True)


def _project(x, W, a_src, a_dst, rb=1000):
    n, f = x.shape
    hd = W.shape[1]
    return pl.pallas_call(
        _proj_body,
        grid=(n // rb,),
        in_specs=[
            pl.BlockSpec((rb, f), lambda i: (i, 0)),
            pl.BlockSpec((f, hd), lambda i: (0, 0)),
            pl.BlockSpec((1, hd), lambda i: (0, 0)),
            pl.BlockSpec((1, hd), lambda i: (0, 0)),
        ],
        out_specs=[
            pl.BlockSpec((rb, hd), lambda i: (i, 0)),
            pl.BlockSpec((rb, 1), lambda i: (i, 0)),
            pl.BlockSpec((rb, 1), lambda i: (i, 0)),
        ],
        out_shape=[
            jax.ShapeDtypeStruct((n, hd), jnp.float32),
            jax.ShapeDtypeStruct((n, 1), jnp.float32),
            jax.ShapeDtypeStruct((n, 1), jnp.float32),
        ],
    )(x, W, a_src.reshape(1, -1), a_dst.reshape(1, -1))


def _mid_body(acc_ref, den_ref, b_ref, w_ref, as_ref, ad_ref,
              h_ref, s_ref, d_ref, *, hin, hd):
    num = acc_ref[0] + acc_ref[1]
    den = jnp.sum(den_ref[...], axis=1, keepdims=True)
    h1 = jnp.maximum(num[:, :hin] / (den + 1e-16) + b_ref[...], 0.0)
    h = jnp.dot(h1, w_ref[...], preferred_element_type=jnp.float32)
    h_ref[...] = h
    s_ref[...] = jnp.sum(h * as_ref[...], axis=1, keepdims=True)
    d_ref[...] = jnp.sum(h * ad_ref[...], axis=1, keepdims=True)


def _mid(acc, den, b, W, a_src, a_dst, n, hin, rb=1000):
    hd = W.shape[1]
    return pl.pallas_call(
        functools.partial(_mid_body, hin=hin, hd=hd),
        grid=(n // rb,),
        in_specs=[
            pl.BlockSpec((2, rb, HP), lambda i: (0, i, 0)),
            pl.BlockSpec((rb, NW), lambda i: (i, 0)),
            pl.BlockSpec((1, hin), lambda i: (0, 0)),
            pl.BlockSpec((hin, hd), lambda i: (0, 0)),
            pl.BlockSpec((1, hd), lambda i: (0, 0)),
            pl.BlockSpec((1, hd), lambda i: (0, 0)),
        ],
        out_specs=[
            pl.BlockSpec((rb, hd), lambda i: (i, 0)),
            pl.BlockSpec((rb, 1), lambda i: (i, 0)),
            pl.BlockSpec((rb, 1), lambda i: (i, 0)),
        ],
        out_shape=[
            jax.ShapeDtypeStruct((n, hd), jnp.float32),
            jax.ShapeDtypeStruct((n, 1), jnp.float32),
            jax.ShapeDtypeStruct((n, 1), jnp.float32),
        ],
    )(acc, den, b.reshape(1, -1), W, a_src.reshape(1, -1), a_dst.reshape(1, -1))


def _fin_body(acc_ref, den_ref, b_ref, z_ref, *, hin):
    num = acc_ref[0] + acc_ref[1]
    den = jnp.sum(den_ref[...], axis=1, keepdims=True)
    z_ref[...] = jnp.tanh(num[:, :hin] / (den + 1e-16) + b_ref[...])


def _fin(acc, den, b, n, hin, rb=1000):
    hp = acc.shape[2]
    return pl.pallas_call(
        functools.partial(_fin_body, hin=hin),
        grid=(n // rb,),
        in_specs=[
            pl.BlockSpec((2, rb, hp), lambda i: (0, i, 0)),
            pl.BlockSpec((rb, NW), lambda i: (i, 0)),
            pl.BlockSpec((1, hin), lambda i: (0, 0)),
        ],
        out_specs=pl.BlockSpec((rb, hin), lambda i: (i, 0)),
        out_shape=jax.ShapeDtypeStruct((n, hin), jnp.float32),
    )(acc, den, b.reshape(1, -1))


def _dec_body(accr_ref, denr_ref, accc_ref, denc_ref, b_ref, o_ref, z_ref,
              *, hin):
    numr = accr_ref[0] + accr_ref[1]
    denr = jnp.sum(denr_ref[...], axis=1, keepdims=True)
    zr = jnp.tanh(numr[:, :hin] / (denr + 1e-16) + b_ref[...])
    numc = accc_ref[0] + accc_ref[1]
    denc = jnp.sum(denc_ref[...], axis=1, keepdims=True)
    zc = jnp.tanh(numc[:, :hin] / (denc + 1e-16) + b_ref[...])
    g = lax.dot_general(zr, zc, (((1,), (1,)), ((), ())),
                        preferred_element_type=jnp.float32)
    o_ref[...] = jax.nn.sigmoid(g)
    z_ref[...] = zr


def _decode(acc, den, b, n, hin, rb=2048, cb=2048):
    hp = acc.shape[2]
    return pl.pallas_call(
        functools.partial(_dec_body, hin=hin),
        grid=(pl.cdiv(n, rb), pl.cdiv(n, cb)),
        in_specs=[
            pl.BlockSpec((2, rb, hp), lambda i, j: (0, i, 0)),
            pl.BlockSpec((rb, NW), lambda i, j: (i, 0)),
            pl.BlockSpec((2, cb, hp), lambda i, j: (0, j, 0)),
            pl.BlockSpec((cb, NW), lambda i, j: (j, 0)),
            pl.BlockSpec((1, hin), lambda i, j: (0, 0)),
        ],
        out_specs=[
            pl.BlockSpec((rb, cb), lambda i, j: (i, j)),
            pl.BlockSpec((rb, hin), lambda i, j: (i, 0)),
        ],
        out_shape=[
            jax.ShapeDtypeStruct((n, n), jnp.float32),
            jax.ShapeDtypeStruct((n, hin), jnp.float32),
        ],
    )(acc, den, acc, den, b.reshape(1, -1))


EC = 64


def _sc_weights(src2d, dst2d, s_pad, d_pad, npad):
    cpt = src2d.shape[0] // NW
    mesh = plsc.VectorSubcoreMesh(
        core_axis_name="c", subcore_axis_name="s", num_cores=NC, num_subcores=NS)

    @functools.partial(
        pl.kernel,
        out_type=[
            jax.ShapeDtypeStruct(src2d.shape, jnp.float32),
            jax.ShapeDtypeStruct((NW, npad), jnp.float32),
        ],
        mesh=mesh,
        compiler_params=pltpu.CompilerParams(needs_layout_passes=False),
        scratch_types=[
            pltpu.VMEM((npad,), jnp.float32),
            pltpu.VMEM((npad,), jnp.float32),
            pltpu.VMEM((npad,), jnp.float32),
            pltpu.VMEM((cpt, EC), jnp.int32),
            pltpu.VMEM((cpt, EC), jnp.int32),
            pltpu.VMEM((cpt, EC), jnp.float32),
        ],
    )
    def k(src_hbm, dst_hbm, sv_hbm, dv_hbm, w_hbm, den_hbm,
          s_v, d_v, dloc, srca, dsta, wa):
        c = lax.axis_index("c")
        sid = lax.axis_index("s")
        wid = c * NS + sid
        pltpu.sync_copy(src_hbm.at[pl.ds(wid * cpt, cpt)], srca)
        pltpu.sync_copy(dst_hbm.at[pl.ds(wid * cpt, cpt)], dsta)
        pltpu.sync_copy(sv_hbm, s_v)
        pltpu.sync_copy(dv_hbm, d_v)
        zero16 = jnp.zeros((L,), jnp.float32)

        def zden(j, carry):
            dloc[pl.ds(j * L, L)] = zero16
            return carry

        lax.fori_loop(0, npad // L, zden, 0)

        def wchunk(i, carry):
            for g in range(EC // L):
                dv = dsta[i, pl.ds(g * L, L)]
                e = (plsc.load_gather(s_v, [srca[i, pl.ds(g * L, L)]])
                     + plsc.load_gather(d_v, [dv]))
                e = jnp.where(e >= 0.0, e, 0.2 * e)
                w = jnp.exp(jnp.minimum(e, 75.0))
                wa[i, pl.ds(g * L, L)] = w
                plsc.addupdate_scatter(dloc, [dv], w)
            return carry

        lax.fori_loop(0, cpt, wchunk, 0)
        pltpu.sync_copy(wa, w_hbm.at[pl.ds(wid * cpt, cpt)])
        pltpu.sync_copy(dloc, den_hbm.at[wid])

    return k(src2d, dst2d, s_pad, d_pad)


def _sc_aggregate(h, src1d, dst2d, w1d, rpad, hp, ec):
    cpt = dst2d.shape[0] // NW
    rpt = rpad // NS
    mesh = plsc.VectorSubcoreMesh(
        core_axis_name="c", subcore_axis_name="s", num_cores=NC, num_subcores=NS)

    @functools.partial(
        pl.kernel,
        out_type=jax.ShapeDtypeStruct((NC, rpad, hp), jnp.float32),
        mesh=mesh,
        compiler_params=pltpu.CompilerParams(needs_layout_passes=False,
                                             use_tc_tiling_on_sc=False),
        scratch_types=[
            pltpu.VMEM((cpt * ec,), jnp.int32),
            pltpu.VMEM((cpt, ec), jnp.int32),
            pltpu.VMEM((2 * ec,), jnp.float32),
            pltpu.VMEM((2, ec, hp), jnp.float32),
            pltpu.VMEM((2, ec, hp), jnp.float32),
            pltpu.VMEM_SHARED((rpad, hp), jnp.float32),
            pltpu.SemaphoreType.DMA,
            pltpu.SemaphoreType.DMA,
            pltpu.SemaphoreType.DMA,
            pltpu.SemaphoreType.DMA,
            pltpu.SemaphoreType.DMA,
            pltpu.SemaphoreType.DMA,
        ],
    )
    def k(h_hbm, src_hbm, dst_hbm, w_hbm, num_hbm,
          srca, dsta, wb, gb, sb, acc, sg0, sg1, ss0, ss1, sw0, sw1):
        c = lax.axis_index("c")
        sid = lax.axis_index("s")
        wid = c * NS + sid
        sgs = (sg0, sg1)
        sss = (ss0, ss1)
        sws = (sw0, sw1)
        zero16 = jnp.zeros((L,), jnp.float32)
        pltpu.sync_copy(src_hbm.at[pl.ds(wid * cpt * ec, cpt * ec)], srca)
        pltpu.sync_copy(dst_hbm.at[pl.ds(wid * cpt, cpt)], dsta)

        base = sid * rpt

        def zrow(j, carry):
            for kk in range(hp // L):
                sb[0, j, pl.ds(kk * L, L)] = zero16
            return carry

        lax.fori_loop(0, ec, zrow, 0)
        off = 0
        while off < rpt:
            step = min(ec, rpt - off)
            pltpu.sync_copy(sb.at[0, pl.ds(0, step)],
                            acc.at[pl.ds(base + off, step)])
            off += step
        plsc.subcore_barrier()

        def g_start(ci, b):
            pltpu.async_copy(h_hbm.at[srca.at[pl.ds(ci * ec, ec)]],
                             gb.at[b], sgs[b])

        def g_wait(b):
            pltpu.make_async_copy(h_hbm.at[srca.at[pl.ds(0, ec)]],
                                  gb.at[b], sgs[b]).wait()

        def w_start(ci, b):
            pltpu.async_copy(w_hbm.at[pl.ds((wid * cpt + ci) * ec, ec)],
                             wb.at[pl.ds(b * ec, ec)], sws[b])

        def w_wait(b):
            pltpu.make_async_copy(w_hbm.at[pl.ds(0, ec)],
                                  wb.at[pl.ds(b * ec, ec)], sws[b]).wait()

        def s_start(ci, b):
            pltpu.async_copy(sb.at[b], acc.at[dsta.at[ci]], sss[b], add=True)

        def s_wait(b):
            pltpu.make_async_copy(sb.at[b], acc.at[dsta.at[0]], sss[b]).wait()

        def scale(b):
            def edge(j, ecarry):
                wspl = plsc.load_gather(
                    wb, [jnp.zeros((L,), jnp.int32) + (j + b * ec)])
                for kk in range(hp // L):
                    sb[b, j, pl.ds(kk * L, L)] = gb[b, j, pl.ds(kk * L, L)] * wspl
                return ecarry

            lax.fori_loop(0, ec, edge, 0)

        for b in range(2):
            g_start(b, b)
            w_start(b, b)
        for b in range(2):
            g_wait(b)
            w_wait(b)
            scale(b)
            s_start(b, b)
            g_start(b + 2, b)
            w_start(b + 2, b)

        def steady(o, carry):
            for b in range(2):
                ci = 2 + 2 * o + b
                g_wait(b)
                w_wait(b)
                s_wait(b)
                scale(b)
                s_start(ci, b)
                g_start(ci + 2, b)
                w_start(ci + 2, b)
            return carry

        lax.fori_loop(0, (cpt - 4) // 2, steady, 0)
        for b in range(2):
            ci = cpt - 2 + b
            g_wait(b)
            w_wait(b)
            s_wait(b)
            scale(b)
            s_start(ci, b)
        for b in range(2):
            s_wait(b)
        plsc.subcore_barrier()

        pltpu.sync_copy(acc.at[pl.ds(base, rpt)],
                        num_hbm.at[c, pl.ds(base, rpt)])

    return k(h, src1d, dst2d, w1d)


def kernel(x, edge_index, W1, a_src1, a_dst1, b1, W2, a_src2, a_dst2, b2):
    n = x.shape[0]
    e = edge_index.shape[1]
    h1d = W1.shape[1]
    h2d = W2.shape[1]
    npad = ((n + 1 + NS * CH - 1) // (NS * CH)) * (NS * CH)
    rpad = ((n + 1 + NS * 8 - 1) // (NS * 8)) * (NS * 8)
    ep = ((e + NW * EC * 2 - 1) // (NW * EC * 2)) * (NW * EC * 2)
    src = jnp.concatenate(
        [edge_index[0], jnp.zeros((ep - e,), jnp.int32)]).reshape(ep // EC, EC)
    dst = jnp.concatenate(
        [edge_index[1], jnp.full((ep - e,), n, jnp.int32)]).reshape(ep // EC, EC)

    h1, s1, d1 = _project(x, W1, a_src1, a_dst1)
    s1p = jnp.pad(s1[:, 0], (0, npad - n))
    d1p = jnp.pad(d1[:, 0], (0, npad - n))
    w1, den1 = _sc_weights(src, dst, s1p, d1p, npad)
    num1 = _sc_aggregate(h1, src.reshape(-1), dst, w1.reshape(-1), rpad,
                         h1d, EC)

    h2, s2, d2 = _mid(num1, den1.T, b1, W2, a_src2, a_dst2,
                      n, h1d)
    s2p = jnp.pad(s2[:, 0], (0, npad - n))
    d2p = jnp.pad(d2[:, 0], (0, npad - n))
    w2, den2 = _sc_weights(src, dst, s2p, d2p, npad)
    num2 = _sc_aggregate(h2, src.reshape(-1), dst.reshape(ep // (2 * EC), 2 * EC),
                         w2.reshape(-1), rpad, h2d, 2 * EC)

    adj, z = _decode(num2, den2.T, b2, n, h2d)
    return (adj, z)


# speedup vs baseline: 15.6022x; 1.0020x over previous
import functools

import jax
import jax.numpy as jnp
from jax import lax
from jax.experimental import pallas as pl
from jax.experimental.pallas import tpu as pltpu
from jax.experimental.pallas import tpu_sc as plsc

NC, NS, L = 2, 16, 16
NW = NC * NS
CH = 128
HP = 128


def _proj_body(x_ref, w_ref, as_ref, ad_ref, h_ref, s_ref, d_ref):
    h = jnp.dot(x_ref[...], w_ref[...], preferred_element_type=jnp.float32)
    h_ref[...] = h
    s_ref[...] = jnp.sum(h * as_ref[...], axis=1, keepdims=True)
    d_ref[...] = jnp.sum(h * ad_ref[...], axis=1, keepdims=True)


def _project(x, W, a_src, a_dst, rb=1000):
    n, f = x.shape
    hd = W.shape[1]
    return pl.pallas_call(
        _proj_body,
        grid=(n // rb,),
        in_specs=[
            pl.BlockSpec((rb, f), lambda i: (i, 0)),
            pl.BlockSpec((f, hd), lambda i: (0, 0)),
            pl.BlockSpec((1, hd), lambda i: (0, 0)),
            pl.BlockSpec((1, hd), lambda i: (0, 0)),
        ],
        out_specs=[
            pl.BlockSpec((rb, hd), lambda i: (i, 0)),
            pl.BlockSpec((rb, 1), lambda i: (i, 0)),
            pl.BlockSpec((rb, 1), lambda i: (i, 0)),
        ],
        out_shape=[
            jax.ShapeDtypeStruct((n, hd), jnp.float32),
            jax.ShapeDtypeStruct((n, 1), jnp.float32),
            jax.ShapeDtypeStruct((n, 1), jnp.float32),
        ],
    )(x, W, a_src.reshape(1, -1), a_dst.reshape(1, -1))


def _mid_body(acc_ref, den_ref, b_ref, w_ref, as_ref, ad_ref,
              h_ref, s_ref, d_ref, *, hin, hd):
    num = acc_ref[0] + acc_ref[1]
    den = jnp.sum(den_ref[...], axis=1, keepdims=True)
    h1 = jnp.maximum(num[:, :hin] / (den + 1e-16) + b_ref[...], 0.0)
    h = jnp.dot(h1, w_ref[...], preferred_element_type=jnp.float32)
    h_ref[...] = h
    s_ref[...] = jnp.sum(h * as_ref[...], axis=1, keepdims=True)
    d_ref[...] = jnp.sum(h * ad_ref[...], axis=1, keepdims=True)


def _mid(acc, den, b, W, a_src, a_dst, n, hin, rb=1000):
    hd = W.shape[1]
    return pl.pallas_call(
        functools.partial(_mid_body, hin=hin, hd=hd),
        grid=(n // rb,),
        in_specs=[
            pl.BlockSpec((2, rb, HP), lambda i: (0, i, 0)),
            pl.BlockSpec((rb, NW), lambda i: (i, 0)),
            pl.BlockSpec((1, hin), lambda i: (0, 0)),
            pl.BlockSpec((hin, hd), lambda i: (0, 0)),
            pl.BlockSpec((1, hd), lambda i: (0, 0)),
            pl.BlockSpec((1, hd), lambda i: (0, 0)),
        ],
        out_specs=[
            pl.BlockSpec((rb, hd), lambda i: (i, 0)),
            pl.BlockSpec((rb, 1), lambda i: (i, 0)),
            pl.BlockSpec((rb, 1), lambda i: (i, 0)),
        ],
        out_shape=[
            jax.ShapeDtypeStruct((n, hd), jnp.float32),
            jax.ShapeDtypeStruct((n, 1), jnp.float32),
            jax.ShapeDtypeStruct((n, 1), jnp.float32),
        ],
    )(acc, den, b.reshape(1, -1), W, a_src.reshape(1, -1), a_dst.reshape(1, -1))


def _fin_body(acc_ref, den_ref, b_ref, z_ref, *, hin):
    num = acc_ref[0] + acc_ref[1]
    den = jnp.sum(den_ref[...], axis=1, keepdims=True)
    z_ref[...] = jnp.tanh(num[:, :hin] / (den + 1e-16) + b_ref[...])


def _fin(acc, den, b, n, hin, rb=1000):
    hp = acc.shape[2]
    return pl.pallas_call(
        functools.partial(_fin_body, hin=hin),
        grid=(n // rb,),
        in_specs=[
            pl.BlockSpec((2, rb, hp), lambda i: (0, i, 0)),
            pl.BlockSpec((rb, NW), lambda i: (i, 0)),
            pl.BlockSpec((1, hin), lambda i: (0, 0)),
        ],
        out_specs=pl.BlockSpec((rb, hin), lambda i: (i, 0)),
        out_shape=jax.ShapeDtypeStruct((n, hin), jnp.float32),
    )(acc, den, b.reshape(1, -1))


def _dec_body(accr_ref, denr_ref, accc_ref, denc_ref, b_ref, o_ref, z_ref,
              *, hin):
    numr = accr_ref[0] + accr_ref[1]
    denr = jnp.sum(denr_ref[...], axis=1, keepdims=True)
    zr = jnp.tanh(numr[:, :hin] / (denr + 1e-16) + b_ref[...])
    numc = accc_ref[0] + accc_ref[1]
    denc = jnp.sum(denc_ref[...], axis=1, keepdims=True)
    zc = jnp.tanh(numc[:, :hin] / (denc + 1e-16) + b_ref[...])
    g = lax.dot_general(zr, zc, (((1,), (1,)), ((), ())),
                        preferred_element_type=jnp.float32)
    o_ref[...] = jax.nn.sigmoid(g)
    z_ref[...] = zr


def _decode(acc, den, b, n, hin, rb=2048, cb=2048):
    hp = acc.shape[2]
    return pl.pallas_call(
        functools.partial(_dec_body, hin=hin),
        grid=(pl.cdiv(n, rb), pl.cdiv(n, cb)),
        in_specs=[
            pl.BlockSpec((2, rb, hp), lambda i, j: (0, i, 0)),
            pl.BlockSpec((rb, NW), lambda i, j: (i, 0)),
            pl.BlockSpec((2, cb, hp), lambda i, j: (0, j, 0)),
            pl.BlockSpec((cb, NW), lambda i, j: (j, 0)),
            pl.BlockSpec((1, hin), lambda i, j: (0, 0)),
        ],
        out_specs=[
            pl.BlockSpec((rb, cb), lambda i, j: (i, j)),
            pl.BlockSpec((rb, hin), lambda i, j: (i, 0)),
        ],
        out_shape=[
            jax.ShapeDtypeStruct((n, n), jnp.float32),
            jax.ShapeDtypeStruct((n, hin), jnp.float32),
        ],
    )(acc, den, acc, den, b.reshape(1, -1))


EC = 64


def _sc_weights(src2d, dst2d, s_pad, d_pad, npad):
    cpt = src2d.shape[0] // NW
    mesh = plsc.VectorSubcoreMesh(
        core_axis_name="c", subcore_axis_name="s", num_cores=NC, num_subcores=NS)

    @functools.partial(
        pl.kernel,
        out_type=[
            jax.ShapeDtypeStruct(src2d.shape, jnp.float32),
            jax.ShapeDtypeStruct((NW, npad), jnp.float32),
        ],
        mesh=mesh,
        compiler_params=pltpu.CompilerParams(needs_layout_passes=False),
        scratch_types=[
            pltpu.VMEM((npad,), jnp.float32),
            pltpu.VMEM((npad,), jnp.float32),
            pltpu.VMEM((npad,), jnp.float32),
            pltpu.VMEM((cpt, EC), jnp.int32),
            pltpu.VMEM((cpt, EC), jnp.int32),
            pltpu.VMEM((cpt, EC), jnp.float32),
        ],
    )
    def k(src_hbm, dst_hbm, sv_hbm, dv_hbm, w_hbm, den_hbm,
          s_v, d_v, dloc, srca, dsta, wa):
        c = lax.axis_index("c")
        sid = lax.axis_index("s")
        wid = c * NS + sid
        pltpu.sync_copy(src_hbm.at[pl.ds(wid * cpt, cpt)], srca)
        pltpu.sync_copy(dst_hbm.at[pl.ds(wid * cpt, cpt)], dsta)
        pltpu.sync_copy(sv_hbm, s_v)
        pltpu.sync_copy(dv_hbm, d_v)
        zero16 = jnp.zeros((L,), jnp.float32)

        def zden(j, carry):
            dloc[pl.ds(j * L, L)] = zero16
            return carry

        lax.fori_loop(0, npad // L, zden, 0)

        def wchunk(i, carry):
            for g in range(EC // L):
                dv = dsta[i, pl.ds(g * L, L)]
                e = (plsc.load_gather(s_v, [srca[i, pl.ds(g * L, L)]])
                     + plsc.load_gather(d_v, [dv]))
                e = jnp.where(e >= 0.0, e, 0.2 * e)
                w = jnp.exp(jnp.minimum(e, 75.0))
                wa[i, pl.ds(g * L, L)] = w
                plsc.addupdate_scatter(dloc, [dv], w)
            return carry

        lax.fori_loop(0, cpt, wchunk, 0)
        pltpu.sync_copy(wa, w_hbm.at[pl.ds(wid * cpt, cpt)])
        pltpu.sync_copy(dloc, den_hbm.at[wid])

    return k(src2d, dst2d, s_pad, d_pad)


def _sc_aggregate(h, src1d, dst2d, w1d, rpad, hp, ec):
    cpt = dst2d.shape[0] // NW
    rpt = rpad // NS
    mesh = plsc.VectorSubcoreMesh(
        core_axis_name="c", subcore_axis_name="s", num_cores=NC, num_subcores=NS)

    @functools.partial(
        pl.kernel,
        out_type=jax.ShapeDtypeStruct((NC, rpad, hp), jnp.float32),
        mesh=mesh,
        compiler_params=pltpu.CompilerParams(needs_layout_passes=False,
                                             use_tc_tiling_on_sc=False),
        scratch_types=[
            pltpu.VMEM((cpt * ec,), jnp.int32),
            pltpu.VMEM((cpt, ec), jnp.int32),
            pltpu.VMEM((2 * ec,), jnp.float32),
            pltpu.VMEM((2, ec, hp), jnp.float32),
            pltpu.VMEM((2, ec, hp), jnp.float32),
            pltpu.VMEM_SHARED((rpad, hp), jnp.float32),
            pltpu.SemaphoreType.DMA,
            pltpu.SemaphoreType.DMA,
            pltpu.SemaphoreType.DMA,
            pltpu.SemaphoreType.DMA,
            pltpu.SemaphoreType.DMA,
            pltpu.SemaphoreType.DMA,
        ],
    )
    def k(h_hbm, src_hbm, dst_hbm, w_hbm, num_hbm,
          srca, dsta, wb, gb, sb, acc, sg0, sg1, ss0, ss1, sw0, sw1):
        c = lax.axis_index("c")
        sid = lax.axis_index("s")
        wid = c * NS + sid
        sgs = (sg0, sg1)
        sss = (ss0, ss1)
        sws = (sw0, sw1)
        zero16 = jnp.zeros((L,), jnp.float32)
        pltpu.sync_copy(src_hbm.at[pl.ds(wid * cpt * ec, cpt * ec)], srca)
        pltpu.sync_copy(dst_hbm.at[pl.ds(wid * cpt, cpt)], dsta)

        base = sid * rpt

        def zrow(j, carry):
            for kk in range(hp // L):
                sb[0, j, pl.ds(kk * L, L)] = zero16
            return carry

        lax.fori_loop(0, ec, zrow, 0)
        off = 0
        while off < rpt:
            step = min(ec, rpt - off)
            pltpu.sync_copy(sb.at[0, pl.ds(0, step)],
                            acc.at[pl.ds(base + off, step)])
            off += step
        plsc.subcore_barrier()

        def g_start(ci, b):
            pltpu.async_copy(h_hbm.at[srca.at[pl.ds(ci * ec, ec)]],
                             gb.at[b], sgs[b])

        def g_wait(b):
            pltpu.make_async_copy(h_hbm.at[srca.at[pl.ds(0, ec)]],
                                  gb.at[b], sgs[b]).wait()

        def w_start(ci, b):
            pltpu.async_copy(w_hbm.at[pl.ds((wid * cpt + ci) * ec, ec)],
                             wb.at[pl.ds(b * ec, ec)], sws[b])

        def w_wait(b):
            pltpu.make_async_copy(w_hbm.at[pl.ds(0, ec)],
                                  wb.at[pl.ds(b * ec, ec)], sws[b]).wait()

        def s_start(ci, b):
            pltpu.async_copy(sb.at[b], acc.at[dsta.at[ci]], sss[b], add=True)

        def s_wait(b):
            pltpu.make_async_copy(sb.at[b], acc.at[dsta.at[0]], sss[b]).wait()

        def scale(b):
            def edge(j, ecarry):
                wspl = plsc.load_gather(
                    wb, [jnp.zeros((L,), jnp.int32) + (j + b * ec)])
                for kk in range(hp // L):
                    sb[b, j, pl.ds(kk * L, L)] = gb[b, j, pl.ds(kk * L, L)] * wspl
                return ecarry

            lax.fori_loop(0, ec, edge, 0)

        for b in range(2):
            g_start(b, b)
            w_start(b, b)
        for b in range(2):
            g_wait(b)
            w_wait(b)
            scale(b)
            s_start(b, b)
            g_start(b + 2, b)
            w_start(b + 2, b)

        def steady(o, carry):
            for b in range(2):
                ci = 2 + 2 * o + b
                g_wait(b)
                w_wait(b)
                s_wait(b)
                scale(b)
                s_start(ci, b)
                g_start(ci + 2, b)
                w_start(ci + 2, b)
            return carry

        lax.fori_loop(0, (cpt - 4) // 2, steady, 0)
        for b in range(2):
            ci = cpt - 2 + b
            g_wait(b)
            w_wait(b)
            s_wait(b)
            scale(b)
            s_start(ci, b)
        for b in range(2):
            s_wait(b)
        plsc.subcore_barrier()

        pltpu.sync_copy(acc.at[pl.ds(base, rpt)],
                        num_hbm.at[c, pl.ds(base, rpt)])

    return k(h, src1d, dst2d, w1d)


def _sc_edge_fused(h, src2d, dst2d, s_pad, d_pad, npad, rpad, hp):
    cpt = src2d.shape[0] // NW
    ec = src2d.shape[1]
    rpt = rpad // NS
    mesh = plsc.VectorSubcoreMesh(
        core_axis_name="c", subcore_axis_name="s", num_cores=NC, num_subcores=NS)

    @functools.partial(
        pl.kernel,
        out_type=[
            jax.ShapeDtypeStruct((NC, rpad, hp), jnp.float32),
            jax.ShapeDtypeStruct((NW, npad), jnp.float32),
        ],
        mesh=mesh,
        compiler_params=pltpu.CompilerParams(needs_layout_passes=False,
                                             use_tc_tiling_on_sc=False),
        scratch_types=[
            pltpu.VMEM((npad,), jnp.float32),
            pltpu.VMEM((npad,), jnp.float32),
            pltpu.VMEM((npad,), jnp.float32),
            pltpu.VMEM((cpt * ec,), jnp.int32),
            pltpu.VMEM((cpt, ec), jnp.int32),
            pltpu.VMEM((cpt, ec), jnp.float32),
            pltpu.VMEM((2, ec, hp), jnp.float32),
            pltpu.VMEM((2, ec, hp), jnp.float32),
            pltpu.VMEM_SHARED((rpad, hp), jnp.float32),
            pltpu.SemaphoreType.DMA,
            pltpu.SemaphoreType.DMA,
            pltpu.SemaphoreType.DMA,
            pltpu.SemaphoreType.DMA,
        ],
    )
    def k(h_hbm, src_hbm, dst_hbm, sv_hbm, dv_hbm, num_hbm, den_hbm,
          s_v, d_v, dloc, srca, dsta, wa, gb, sb, acc, sg0, sg1, ss0, ss1):
        c = lax.axis_index("c")
        sid = lax.axis_index("s")
        wid = c * NS + sid
        sgs = (sg0, sg1)
        sss = (ss0, ss1)
        zero16 = jnp.zeros((L,), jnp.float32)
        pltpu.sync_copy(src_hbm.at[pl.ds(wid * cpt * ec, cpt * ec)], srca)
        pltpu.sync_copy(dst_hbm.at[pl.ds(wid * cpt, cpt)], dsta)
        pltpu.sync_copy(sv_hbm, s_v)
        pltpu.sync_copy(dv_hbm, d_v)

        def zden(j, carry):
            dloc[pl.ds(j * L, L)] = zero16
            return carry

        lax.fori_loop(0, npad // L, zden, 0)

        def wchunk(i, carry):
            for g in range(ec // L):
                dv = dsta[i, pl.ds(g * L, L)]
                e = (plsc.load_gather(s_v, [srca[pl.ds(i * ec + g * L, L)]])
                     + plsc.load_gather(d_v, [dv]))
                e = jnp.where(e >= 0.0, e, 0.2 * e)
                w = jnp.exp(jnp.minimum(e, 75.0))
                wa[i, pl.ds(g * L, L)] = w
                plsc.addupdate_scatter(dloc, [dv], w)
            return carry

        lax.fori_loop(0, cpt, wchunk, 0)
        pltpu.sync_copy(dloc, den_hbm.at[wid])

        base = sid * rpt

        def zrow(j, carry):
            for kk in range(hp // L):
                sb[0, j, pl.ds(kk * L, L)] = zero16
            return carry

        lax.fori_loop(0, ec, zrow, 0)
        off = 0
        while off < rpt:
            step = min(ec, rpt - off)
            pltpu.sync_copy(sb.at[0, pl.ds(0, step)],
                            acc.at[pl.ds(base + off, step)])
            off += step
        plsc.subcore_barrier()

        def g_start(ci, b):
            pltpu.async_copy(h_hbm.at[srca.at[pl.ds(ci * ec, ec)]],
                             gb.at[b], sgs[b])

        def g_wait(b):
            pltpu.make_async_copy(h_hbm.at[srca.at[pl.ds(0, ec)]],
                                  gb.at[b], sgs[b]).wait()

        def s_start(ci, b):
            pltpu.async_copy(sb.at[b], acc.at[dsta.at[ci]], sss[b], add=True)

        def s_wait(b):
            pltpu.make_async_copy(sb.at[b], acc.at[dsta.at[0]], sss[b]).wait()

        def scale(ci, b):
            cspl = jnp.zeros((L,), jnp.int32) + ci

            def edge(j, ecarry):
                wspl = plsc.load_gather(
                    wa, [cspl, jnp.zeros((L,), jnp.int32) + j])
                for kk in range(hp // L):
                    sb[b, j, pl.ds(kk * L, L)] = gb[b, j, pl.ds(kk * L, L)] * wspl
                return ecarry

            lax.fori_loop(0, ec, edge, 0)

        g_start(0, 0)
        g_start(1, 1)
        for b in range(2):
            g_wait(b)
            scale(b, b)
            s_start(b, b)
            g_start(b + 2, b)

        def steady(o, carry):
            for b in range(2):
                ci = 2 + 2 * o + b
                g_wait(b)
                s_wait(b)
                scale(ci, b)
                s_start(ci, b)
                g_start(ci + 2, b)
            return carry

        lax.fori_loop(0, (cpt - 4) // 2, steady, 0)
        for b in range(2):
            ci = cpt - 2 + b
            g_wait(b)
            s_wait(b)
            scale(ci, b)
            s_start(ci, b)
        for b in range(2):
            s_wait(b)
        plsc.subcore_barrier()

        pltpu.sync_copy(acc.at[pl.ds(base, rpt)],
                        num_hbm.at[c, pl.ds(base, rpt)])

    return k(h, src2d.reshape(-1), dst2d, s_pad, d_pad)


def kernel(x, edge_index, W1, a_src1, a_dst1, b1, W2, a_src2, a_dst2, b2):
    n = x.shape[0]
    e = edge_index.shape[1]
    h1d = W1.shape[1]
    h2d = W2.shape[1]
    npad = ((n + 1 + NS * CH - 1) // (NS * CH)) * (NS * CH)
    rpad = ((n + 1 + NS * 8 - 1) // (NS * 8)) * (NS * 8)
    ep = ((e + NW * EC * 2 - 1) // (NW * EC * 2)) * (NW * EC * 2)
    src = jnp.concatenate(
        [edge_index[0], jnp.zeros((ep - e,), jnp.int32)]).reshape(ep // EC, EC)
    dst = jnp.concatenate(
        [edge_index[1], jnp.full((ep - e,), n, jnp.int32)]).reshape(ep // EC, EC)

    h1, s1, d1 = _project(x, W1, a_src1, a_dst1)
    s1p = jnp.pad(s1[:, 0], (0, npad - n))
    d1p = jnp.pad(d1[:, 0], (0, npad - n))
    w1, den1 = _sc_weights(src, dst, s1p, d1p, npad)
    num1 = _sc_aggregate(h1, src.reshape(-1), dst, w1.reshape(-1), rpad,
                         h1d, EC)

    h2, s2, d2 = _mid(num1, den1.T, b1, W2, a_src2, a_dst2,
                      n, h1d)
    s2p = jnp.pad(s2[:, 0], (0, npad - n))
    d2p = jnp.pad(d2[:, 0], (0, npad - n))
    num2, den2 = _sc_edge_fused(
        h2, src.reshape(ep // (2 * EC), 2 * EC),
        dst.reshape(ep // (2 * EC), 2 * EC), s2p, d2p, npad, rpad, h2d)

    adj, z = _decode(num2, den2.T, b2, n, h2d)
    return (adj, z)
